# Initial kernel scaffold; baseline (speedup 1.0000x reference)
import numpy as np
import jax
import jax.numpy as jnp
from jax import lax
from jax.experimental import pallas as pl
from jax.experimental.pallas import tpu as pltpu
from jax.experimental.pallas import tpu_sc as plsc

N = 10000
E = 320000
D = 128
HEADS = 4
HD = D // HEADS
G = 100
NPG = N // G
NC = 2
NS = 16
NW = NC * NS
EPW = E // NW
CK = 125
NCH = EPW // CK
RPT = N // NS
DEGW = 16
ZR = 25


def _knots(grid_size, order):
    h = np.float32(2.0 / grid_size)
    g = (np.arange(-order, grid_size + order + 1, dtype=np.float32) * h
         - np.float32(1.0))
    return [float(v) for v in g]


K3 = _knots(4, 3)
L1 = _knots(2, 1)


def _sc_deg_body(row_hbm, out_hbm, idx_v, ones_v, zero_v, tab):
    c = lax.axis_index("c")
    s = lax.axis_index("s")
    wid = c * NS + s
    pltpu.sync_copy(row_hbm.at[wid], idx_v)

    def _fill_ones(i, _):
        ones_v[i, :] = jnp.full((DEGW,), 1.0, jnp.float32)
        return 0
    lax.fori_loop(0, CK, _fill_ones, 0)

    def _fill_zero(i, _):
        zero_v[i, :] = jnp.zeros((DEGW,), jnp.float32)
        return 0
    lax.fori_loop(0, ZR, _fill_zero, 0)

    def _zero_stripe(i, _):
        pltpu.sync_copy(zero_v, tab.at[pl.ds(s * RPT + i * ZR, ZR)])
        return 0
    lax.fori_loop(0, RPT // ZR, _zero_stripe, 0)
    plsc.subcore_barrier()

    def _scat(j, _):
        pltpu.sync_copy(ones_v, tab.at[idx_v.at[j]], add=True)
        return 0
    lax.fori_loop(0, NCH, _scat, 0)
    plsc.subcore_barrier()

    pltpu.sync_copy(tab.at[pl.ds(s * RPT, RPT)],
                    out_hbm.at[c, pl.ds(s * RPT, RPT)])


def _sc_scat_body(y_hbm, row_hbm, col_hbm, out_hbm,
                  ridx_v, cidx_v, buf0, buf1, zero_v, acc, sem0, sem1):
    c = lax.axis_index("c")
    s = lax.axis_index("s")
    wid = c * NS + s
    pltpu.sync_copy(row_hbm.at[wid], ridx_v)
    pltpu.sync_copy(col_hbm.at[wid], cidx_v)

    def _fill_zero(t, _):
        i = t // (D // 16)
        k = t % (D // 16)
        zero_v[i, pl.ds(k * 16, 16)] = jnp.zeros((16,), jnp.float32)
        return 0
    lax.fori_loop(0, ZR * (D // 16), _fill_zero, 0)

    def _zero_stripe(i, _):
        pltpu.sync_copy(zero_v, acc.at[pl.ds(s * RPT + i * ZR, ZR)])
        return 0
    lax.fori_loop(0, RPT // ZR, _zero_stripe, 0)
    plsc.subcore_barrier()

    pltpu.async_copy(y_hbm.at[ridx_v.at[0]], buf0, sem0)

    def _step(j, _):
        pltpu.async_copy(y_hbm.at[ridx_v.at[2 * j + 1]], buf1, sem1)
        pltpu.make_async_copy(y_hbm.at[ridx_v.at[2 * j]], buf0, sem0).wait()
        pltpu.sync_copy(buf0, acc.at[cidx_v.at[2 * j]], add=True)
        pltpu.async_copy(y_hbm.at[ridx_v.at[(2 * j + 2) % NCH]], buf0, sem0)
        pltpu.make_async_copy(y_hbm.at[ridx_v.at[2 * j + 1]], buf1, sem1).wait()
        pltpu.sync_copy(buf1, acc.at[cidx_v.at[2 * j + 1]], add=True)
        return 0
    lax.fori_loop(0, NCH // 2, _step, 0)
    pltpu.make_async_copy(y_hbm.at[ridx_v.at[0]], buf0, sem0).wait()
    plsc.subcore_barrier()

    pltpu.sync_copy(acc.at[pl.ds(s * RPT, RPT)],
                    out_hbm.at[c, pl.ds(s * RPT, RPT)])


_sc_mesh = plsc.VectorSubcoreMesh(core_axis_name="c", subcore_axis_name="s")

_deg_call = pl.kernel(
    _sc_deg_body,
    out_type=jax.ShapeDtypeStruct((NC, N, DEGW), jnp.float32),
    mesh=_sc_mesh,
    scratch_types=[
        pltpu.VMEM((NCH, CK), jnp.int32),
        pltpu.VMEM((CK, DEGW), jnp.float32),
        pltpu.VMEM((ZR, DEGW), jnp.float32),
        pltpu.VMEM_SHARED((N, DEGW), jnp.float32),
    ],
)

_scat_call = pl.kernel(
    _sc_scat_body,
    out_type=jax.ShapeDtypeStruct((NC, N, D), jnp.float32),
    mesh=_sc_mesh,
    scratch_types=[
        pltpu.VMEM((NCH, CK), jnp.int32),
        pltpu.VMEM((NCH, CK), jnp.int32),
        pltpu.VMEM((CK, D), jnp.float32),
        pltpu.VMEM((CK, D), jnp.float32),
        pltpu.VMEM((ZR, D), jnp.float32),
        pltpu.VMEM_SHARED((N, D), jnp.float32),
        pltpu.SemaphoreType.DMA,
        pltpu.SemaphoreType.DMA,
    ],
)


RB = 1250


def _scale_body(x_ref, dp_ref, y_ref, dv_ref):
    d = dp_ref[0] + dp_ref[1]
    deg = d[:, 0:1] + 1.0
    dinv = lax.rsqrt(deg)
    y_ref[...] = x_ref[...] * dinv
    dv_ref[...] = jnp.broadcast_to(dinv, (RB, DEGW))


def _dgT(a, b):
    return lax.dot_general(a, b, (((1,), (1,)), ((), ())),
                           precision=lax.Precision.HIGHEST,
                           preferred_element_type=jnp.float32)


def _dg(a, b):
    return lax.dot_general(a, b, (((1,), (0,)), ((), ())),
                           precision=lax.Precision.HIGHEST,
                           preferred_element_type=jnp.float32)


def _bsplines(t, knots, order):
    nb = len(knots) - 1
    b = [((t >= knots[i]) & (t < knots[i + 1])).astype(t.dtype)
         for i in range(nb)]
    for j in range(1, order + 1):
        b = [(t - knots[i]) / (knots[i + j] - knots[i]) * b[i]
             + (knots[i + j + 1] - t) / (knots[i + j + 1] - knots[i + 1])
             * b[i + 1]
             for i in range(nb - j)]
    return b


def _kan_in128(t, bw_ref, sw_ref, knots, order, act):
    out = _dgT(act(t), bw_ref[...])
    for i, bi in enumerate(_bsplines(t, knots, order)):
        out = out + _dgT(bi, sw_ref[i])
    return out


def _kan_in16(t, bwt_ref, swt_ref, knots, order, act):
    out = _dg(act(t), bwt_ref[...])
    for i, bi in enumerate(_bsplines(t, knots, order)):
        out = out + _dg(bi, swt_ref[i])
    return out


def _ln(t):
    mu = jnp.mean(t, axis=-1, keepdims=True)
    tc = t - mu
    var = jnp.mean(tc * tc, axis=-1, keepdims=True)
    return tc / jnp.sqrt(var + 1e-5)


_SQRT_HD = float(np.sqrt(np.float32(HD)))


def _attn(q, k, v):
    parts = []
    for h in range(HEADS):
        qh = q[:, h * HD:(h + 1) * HD]
        kh = k[:, h * HD:(h + 1) * HD]
        vh = v[:, h * HD:(h + 1) * HD]
        sc = _dgT(qh, kh) / _SQRT_HD
        m = jnp.max(sc, axis=-1, keepdims=True)
        e = jnp.exp(sc - m)
        p = e / jnp.sum(e, axis=-1, keepdims=True)
        parts.append(_dg(p, vh))
    return jnp.concatenate(parts, axis=1)


def _silu(t):
    return t * (1.0 / (1.0 + jnp.exp(-t)))


def _ident(t):
    return t


def _fwd_body(x_ref, s_ref, dv_ref,
              qb, qs, kb, ks, vb, vs, ob, osw,
              l1b, l1s, l2b, l2s, f1b, f1s, f2b, f2s,
              o_ref):
    x = x_ref[0]
    sagg = s_ref[0, 0] + s_ref[1, 0]
    dinv = dv_ref[0][:, 0:1]
    agg = dinv * sagg + (dinv * dinv) * x

    t1 = _kan_in128(agg, l1b, l1s, K3, 3, _silu)
    h_local = _kan_in16(t1, l2b, l2s, K3, 3, _silu)

    q = _kan_in128(x, qb, qs, L1, 1, _ident)
    k = _kan_in128(x, kb, ks, L1, 1, _ident)
    v = _kan_in128(x, vb, vs, L1, 1, _ident)
    ctx = _attn(q, k, v)
    h_attn = _kan_in128(ctx, ob, osw, L1, 1, _ident)

    h1 = _ln(x + h_local)
    h2 = _ln(x + h_attn)
    h = h1 + h2
    t2 = _kan_in128(h, f1b, f1s, K3, 3, _silu)
    ffn = _kan_in16(t2, f2b, f2s, K3, 3, _silu)
    o_ref[0] = _ln(h + ffn)


def _whole(shape):
    nd = len(shape)
    return pl.BlockSpec(shape, lambda i, _nd=nd: (0,) * _nd)


_scale_call = pl.pallas_call(
    _scale_body,
    grid=(N // RB,),
    in_specs=[
        pl.BlockSpec((RB, D), lambda i: (i, 0)),
        pl.BlockSpec((NC, RB, DEGW), lambda i: (0, i, 0)),
    ],
    out_specs=[
        pl.BlockSpec((RB, D), lambda i: (i, 0)),
        pl.BlockSpec((RB, DEGW), lambda i: (i, 0)),
    ],
    out_shape=[
        jax.ShapeDtypeStruct((N, D), jnp.float32),
        jax.ShapeDtypeStruct((N, DEGW), jnp.float32),
    ],
)


def _fwd_in_specs():
    specs = [
        pl.BlockSpec((1, NPG, D), lambda i: (i, 0, 0)),
        pl.BlockSpec((NC, 1, NPG, D), lambda i: (0, i, 0, 0)),
        pl.BlockSpec((1, NPG, DEGW), lambda i: (i, 0, 0)),
    ]
    for shape in [(D, D), (3, D, D)] * 4:
        specs.append(_whole(shape))
    for shape in [(16, D), (7, 16, D), (16, D), (7, 16, D)] * 2:
        specs.append(_whole(shape))
    return specs


_fwd_call = pl.pallas_call(
    _fwd_body,
    grid=(G,),
    in_specs=_fwd_in_specs(),
    out_specs=pl.BlockSpec((1, NPG, D), lambda i: (i, 0, 0)),
    out_shape=jax.ShapeDtypeStruct((G, NPG, D), jnp.float32),
)


def _prep_weights(params):
    out = []
    for name in ("q", "k", "v", "o"):
        bw, sw = params[name]
        out += [bw, jnp.transpose(sw, (2, 0, 1))]
    for name in ("local", "ffn"):
        (b1, s1), (b2, s2) = params[name]
        out += [b1, jnp.transpose(s1, (2, 0, 1)),
                b2.T, jnp.transpose(s2, (2, 1, 0))]
    return out


def kernel(x, edge_index, batch, params):
    del batch
    ei = edge_index.astype(jnp.int32)
    row = ei[0].reshape(NW, NCH, CK)
    col = ei[1].reshape(NW, NCH, CK)
    degp = _deg_call(row)
    y, dv = _scale_call(x, degp)
    sp = _scat_call(y, row, col)
    w = _prep_weights(params)
    out = _fwd_call(x.reshape(G, NPG, D),
                    sp.reshape(NC, G, NPG, D),
                    dv.reshape(G, NPG, DEGW),
                    *w)
    return out.reshape(N, D)


# speedup vs baseline: 5.5580x; 5.5580x over previous
import numpy as np
import jax
import jax.numpy as jnp
from jax import lax
from jax.experimental import pallas as pl
from jax.experimental.pallas import tpu as pltpu
from jax.experimental.pallas import tpu_sc as plsc

N = 10000
E = 320000
D = 128
HEADS = 4
HD = D // HEADS
G = 100
NPG = N // G
NC = 2
NS = 16
NW = NC * NS
EPW = E // NW
CK = 100
NCH = EPW // CK


def _knots(grid_size, order):
    h = np.float32(2.0 / grid_size)
    g = (np.arange(-order, grid_size + order + 1, dtype=np.float32) * h
         - np.float32(1.0))
    return [float(v) for v in g]


K3 = _knots(4, 3)
L1 = _knots(2, 1)


def _sc_deg_body(row_hbm, zero_hbm, out_hbm, idx_v, ones_v, tab):
    c = lax.axis_index("c")
    s = lax.axis_index("s")
    wid = c * NS + s
    pltpu.sync_copy(row_hbm.at[wid], idx_v)
    ones_v[0, :] = jnp.ones((CK,), jnp.float32)

    @pl.when(s == 0)
    def _zero():
        pltpu.sync_copy(zero_hbm, tab)
    plsc.subcore_barrier()

    def _scat(j, _):
        pltpu.sync_copy(ones_v, tab.at[idx_v.at[j]], add=True)
        return 0
    lax.fori_loop(0, NCH, _scat, 0)
    plsc.subcore_barrier()

    @pl.when(s == 0)
    def _out():
        pltpu.sync_copy(tab, out_hbm.at[c])


def _sc_scat_body(y_hbm, row_hbm, col_hbm, zero_hbm, out_hbm,
                  ridx_v, cidx_v, buf, acc, sem):
    c = lax.axis_index("c")
    s = lax.axis_index("s")
    wid = c * NS + s
    pltpu.sync_copy(row_hbm.at[wid], ridx_v)
    pltpu.sync_copy(col_hbm.at[wid], cidx_v)

    @pl.when(s == 0)
    def _zero():
        pltpu.sync_copy(zero_hbm, acc.at[0])
    plsc.subcore_barrier()

    def _step(j, _):
        pltpu.async_copy(y_hbm.at[ridx_v.at[j]], buf, sem).wait()
        pltpu.sync_copy(buf, acc.at[cidx_v.at[j]], add=True)
        return 0
    lax.fori_loop(0, NCH, _step, 0)
    plsc.subcore_barrier()

    @pl.when(s == 0)
    def _out():
        pltpu.sync_copy(acc.at[0], out_hbm.at[c])


_SC_CALLS = {}


def _sc_calls():
    if not _SC_CALLS:
        mesh = plsc.VectorSubcoreMesh(core_axis_name="c",
                                      subcore_axis_name="s")
        _SC_CALLS["deg"] = pl.kernel(
            _sc_deg_body,
            out_type=jax.ShapeDtypeStruct((NC, 1, N), jnp.float32),
            mesh=mesh,
            scratch_types=[
                pltpu.VMEM((NCH, 1, CK), jnp.int32),
                pltpu.VMEM((1, CK), jnp.float32),
                pltpu.VMEM_SHARED((1, N), jnp.float32),
            ],
        )
        _SC_CALLS["scat"] = pl.kernel(
            _sc_scat_body,
            out_type=jax.ShapeDtypeStruct((NC, N, D), jnp.float32),
            mesh=mesh,
            scratch_types=[
                pltpu.VMEM((NCH, 1, CK), jnp.int32),
                pltpu.VMEM((NCH, 1, CK), jnp.int32),
                pltpu.VMEM((1, CK, D), jnp.float32),
                pltpu.VMEM_SHARED((1, N, D), jnp.float32),
                pltpu.SemaphoreType.DMA,
            ],
        )
    return _SC_CALLS["deg"], _SC_CALLS["scat"]


RB = 1000


def _scale_body(x_ref, dp_ref, y_ref, dv_ref):
    d = dp_ref[0] + dp_ref[1]
    dinv = lax.rsqrt(d + 1.0)
    y_ref[...] = x_ref[...] * dinv
    dv_ref[...] = dinv


def _dgT(a, b):
    return lax.dot_general(a, b, (((1,), (1,)), ((), ())),
                           precision=lax.Precision.HIGHEST,
                           preferred_element_type=jnp.float32)


def _dg(a, b):
    return lax.dot_general(a, b, (((1,), (0,)), ((), ())),
                           precision=lax.Precision.HIGHEST,
                           preferred_element_type=jnp.float32)


def _bsplines(t, knots, order):
    nb = len(knots) - 1
    b = [((t >= knots[i]) & (t < knots[i + 1])).astype(t.dtype)
         for i in range(nb)]
    for j in range(1, order + 1):
        b = [(t - knots[i]) / (knots[i + j] - knots[i]) * b[i]
             + (knots[i + j + 1] - t) / (knots[i + j + 1] - knots[i + 1])
             * b[i + 1]
             for i in range(nb - j)]
    return b


def _kan_in128(t, bw_ref, sw_ref, knots, order, act):
    out = _dgT(act(t), bw_ref[...])
    for i, bi in enumerate(_bsplines(t, knots, order)):
        out = out + _dgT(bi, sw_ref[i])
    return out


def _kan_in16(t, bwt_ref, swt_ref, knots, order, act):
    out = _dg(act(t), bwt_ref[...])
    for i, bi in enumerate(_bsplines(t, knots, order)):
        out = out + _dg(bi, swt_ref[i])
    return out


def _ln(t):
    mu = jnp.mean(t, axis=-1, keepdims=True)
    tc = t - mu
    var = jnp.mean(tc * tc, axis=-1, keepdims=True)
    return tc / jnp.sqrt(var + 1e-5)


_SQRT_HD = float(np.sqrt(np.float32(HD)))


def _attn(q, k, v):
    parts = []
    for h in range(HEADS):
        qh = q[:, h * HD:(h + 1) * HD]
        kh = k[:, h * HD:(h + 1) * HD]
        vh = v[:, h * HD:(h + 1) * HD]
        sc = _dgT(qh, kh) / _SQRT_HD
        m = jnp.max(sc, axis=-1, keepdims=True)
        e = jnp.exp(sc - m)
        p = e / jnp.sum(e, axis=-1, keepdims=True)
        parts.append(_dg(p, vh))
    return jnp.concatenate(parts, axis=1)


def _silu(t):
    return t * (1.0 / (1.0 + jnp.exp(-t)))


def _ident(t):
    return t


def _fwd_body(x_ref, s_ref, dv_ref,
              qb, qs, kb, ks, vb, vs, ob, osw,
              l1b, l1s, l2b, l2s, f1b, f1s, f2b, f2s,
              o_ref):
    x = x_ref[0]
    sagg = s_ref[0, 0] + s_ref[1, 0]
    dinv = dv_ref[0]
    agg = dinv * sagg + (dinv * dinv) * x

    t1 = _kan_in128(agg, l1b, l1s, K3, 3, _silu)
    h_local = _kan_in16(t1, l2b, l2s, K3, 3, _silu)

    q = _kan_in128(x, qb, qs, L1, 1, _ident)
    k = _kan_in128(x, kb, ks, L1, 1, _ident)
    v = _kan_in128(x, vb, vs, L1, 1, _ident)
    ctx = _attn(q, k, v)
    h_attn = _kan_in128(ctx, ob, osw, L1, 1, _ident)

    h1 = _ln(x + h_local)
    h2 = _ln(x + h_attn)
    h = h1 + h2
    t2 = _kan_in128(h, f1b, f1s, K3, 3, _silu)
    ffn = _kan_in16(t2, f2b, f2s, K3, 3, _silu)
    o_ref[0] = _ln(h + ffn)


def _whole(shape):
    nd = len(shape)
    return pl.BlockSpec(shape, lambda i, _nd=nd: (0,) * _nd)


_scale_call = pl.pallas_call(
    _scale_body,
    grid=(N // RB,),
    in_specs=[
        pl.BlockSpec((RB, D), lambda i: (i, 0)),
        pl.BlockSpec((NC, RB, 1), lambda i: (0, i, 0)),
    ],
    out_specs=[
        pl.BlockSpec((RB, D), lambda i: (i, 0)),
        pl.BlockSpec((RB, 1), lambda i: (i, 0)),
    ],
    out_shape=[
        jax.ShapeDtypeStruct((N, D), jnp.float32),
        jax.ShapeDtypeStruct((N, 1), jnp.float32),
    ],
)


def _fwd_in_specs():
    specs = [
        pl.BlockSpec((1, NPG, D), lambda i: (i, 0, 0)),
        pl.BlockSpec((NC, 1, NPG, D), lambda i: (0, i, 0, 0)),
        pl.BlockSpec((1, NPG, 1), lambda i: (i, 0, 0)),
    ]
    for shape in [(D, D), (3, D, D)] * 4:
        specs.append(_whole(shape))
    for shape in [(16, D), (7, 16, D), (16, D), (7, 16, D)] * 2:
        specs.append(_whole(shape))
    return specs


_fwd_call = pl.pallas_call(
    _fwd_body,
    grid=(G,),
    in_specs=_fwd_in_specs(),
    out_specs=pl.BlockSpec((1, NPG, D), lambda i: (i, 0, 0)),
    out_shape=jax.ShapeDtypeStruct((G, NPG, D), jnp.float32),
)


def _prep_weights(params):
    out = []
    for name in ("q", "k", "v", "o"):
        bw, sw = params[name]
        out += [bw, jnp.transpose(sw, (2, 0, 1))]
    for name in ("local", "ffn"):
        (b1, s1), (b2, s2) = params[name]
        out += [b1, jnp.transpose(s1, (2, 0, 1)),
                b2.T, jnp.transpose(s2, (2, 1, 0))]
    return out


def kernel(x, edge_index, batch, params):
    del batch
    ei = edge_index.astype(jnp.int32)
    row = ei[0].reshape(NW, NCH, 1, CK)
    col = ei[1].reshape(NW, NCH, 1, CK)
    z1 = jnp.zeros((1, N), jnp.float32)
    znd = jnp.zeros((N, D), jnp.float32)
    deg_call, scat_call = _sc_calls()
    degp = deg_call(row, z1)
    degt = jnp.swapaxes(degp, 1, 2)
    y, dv = _scale_call(x, degt)
    sp = scat_call(y.reshape(1, N, D), row, col, znd)
    w = _prep_weights(params)
    out = _fwd_call(x.reshape(G, NPG, D),
                    sp.reshape(NC, G, NPG, D),
                    dv.reshape(G, NPG, 1),
                    *w)
    return out.reshape(N, D)


# speedup vs baseline: 7.4725x; 1.3444x over previous
import numpy as np
import jax
import jax.numpy as jnp
from jax import lax
from jax.experimental import pallas as pl
from jax.experimental.pallas import tpu as pltpu
from jax.experimental.pallas import tpu_sc as plsc

N = 10000
E = 320000
D = 128
HEADS = 4
HD = D // HEADS
G = 100
NPG = N // G
NC = 2
NS = 16
NW = NC * NS
EPW = E // NW
CK = 100
NCH = EPW // CK


def _knots(grid_size, order):
    h = np.float32(2.0 / grid_size)
    g = (np.arange(-order, grid_size + order + 1, dtype=np.float32) * h
         - np.float32(1.0))
    return [float(v) for v in g]


K3 = _knots(4, 3)
L1 = _knots(2, 1)


def _sc_deg_body(row_hbm, zero_hbm, out_hbm, idx_v, ones_v, tab):
    c = lax.axis_index("c")
    s = lax.axis_index("s")
    wid = c * NS + s
    pltpu.sync_copy(row_hbm.at[wid], idx_v)
    ones_v[0, :] = jnp.ones((CK,), jnp.float32)

    @pl.when(s == 0)
    def _zero():
        pltpu.sync_copy(zero_hbm, tab)
    plsc.subcore_barrier()

    def _scat(j, _):
        pltpu.sync_copy(ones_v, tab.at[idx_v.at[j]], add=True)
        return 0
    lax.fori_loop(0, NCH, _scat, 0)
    plsc.subcore_barrier()

    @pl.when(s == 0)
    def _out():
        pltpu.sync_copy(tab, out_hbm.at[c])


def _sc_scat_body(y_hbm, row_hbm, col_hbm, zero_hbm, out_hbm,
                  ridx_v, cidx_v, buf, acc, sem):
    c = lax.axis_index("c")
    s = lax.axis_index("s")
    wid = c * NS + s
    pltpu.sync_copy(row_hbm.at[wid], ridx_v)
    pltpu.sync_copy(col_hbm.at[wid], cidx_v)

    @pl.when(s == 0)
    def _zero():
        pltpu.sync_copy(zero_hbm, acc.at[0])
    plsc.subcore_barrier()

    def _step(j, _):
        pltpu.async_copy(y_hbm.at[ridx_v.at[j]], buf, sem).wait()
        pltpu.sync_copy(buf, acc.at[cidx_v.at[j]], add=True)
        return 0
    lax.fori_loop(0, NCH, _step, 0)
    plsc.subcore_barrier()

    @pl.when(s == 0)
    def _out():
        pltpu.sync_copy(acc.at[0], out_hbm.at[c])


_SC_CALLS = {}


def _sc_calls():
    if not _SC_CALLS:
        mesh = plsc.VectorSubcoreMesh(core_axis_name="c",
                                      subcore_axis_name="s")
        _SC_CALLS["deg"] = pl.kernel(
            _sc_deg_body,
            out_type=jax.ShapeDtypeStruct((NC, 1, N), jnp.float32),
            mesh=mesh,
            scratch_types=[
                pltpu.VMEM((NCH, 1, CK), jnp.int32),
                pltpu.VMEM((1, CK), jnp.float32),
                pltpu.VMEM_SHARED((1, N), jnp.float32),
            ],
        )
        _SC_CALLS["scat"] = pl.kernel(
            _sc_scat_body,
            out_type=jax.ShapeDtypeStruct((NC, N, D), jnp.float32),
            mesh=mesh,
            scratch_types=[
                pltpu.VMEM((NCH, 1, CK), jnp.int32),
                pltpu.VMEM((NCH, 1, CK), jnp.int32),
                pltpu.VMEM((1, CK, D), jnp.float32),
                pltpu.VMEM_SHARED((1, N, D), jnp.float32),
                pltpu.SemaphoreType.DMA,
            ],
        )
    return _SC_CALLS["deg"], _SC_CALLS["scat"]


RB = 1000


def _scale_body(x_ref, dp_ref, y_ref, dv_ref):
    d = dp_ref[0] + dp_ref[1]
    dinv = lax.rsqrt(d + 1.0)
    y_ref[...] = x_ref[...] * dinv
    dv_ref[...] = dinv


def _dgT(a, b):
    return lax.dot_general(a, b, (((1,), (1,)), ((), ())),
                           precision=lax.Precision.HIGHEST,
                           preferred_element_type=jnp.float32)


def _dg(a, b):
    return lax.dot_general(a, b, (((1,), (0,)), ((), ())),
                           precision=lax.Precision.HIGHEST,
                           preferred_element_type=jnp.float32)


def _bsplines(t, knots, order):
    nb = len(knots) - 1
    b = [((t >= knots[i]) & (t < knots[i + 1])).astype(t.dtype)
         for i in range(nb)]
    for j in range(1, order + 1):
        b = [(t - knots[i]) / (knots[i + j] - knots[i]) * b[i]
             + (knots[i + j + 1] - t) / (knots[i + j + 1] - knots[i + 1])
             * b[i + 1]
             for i in range(nb - j)]
    return b


def _kan_in128(t, bw_ref, sw_ref, knots, order, act):
    out = _dgT(act(t), bw_ref[...])
    for i, bi in enumerate(_bsplines(t, knots, order)):
        out = out + _dgT(bi, sw_ref[i])
    return out


def _kan_in16(t, bwt_ref, swt_ref, knots, order, act):
    out = _dg(act(t), bwt_ref[...])
    for i, bi in enumerate(_bsplines(t, knots, order)):
        out = out + _dg(bi, swt_ref[i])
    return out


def _ln(t):
    mu = jnp.mean(t, axis=-1, keepdims=True)
    tc = t - mu
    var = jnp.mean(tc * tc, axis=-1, keepdims=True)
    return tc / jnp.sqrt(var + 1e-5)


_SQRT_HD = float(np.sqrt(np.float32(HD)))


def _attn(q, k, v):
    parts = []
    for h in range(HEADS):
        qh = q[:, h * HD:(h + 1) * HD]
        kh = k[:, h * HD:(h + 1) * HD]
        vh = v[:, h * HD:(h + 1) * HD]
        sc = _dgT(qh, kh) / _SQRT_HD
        m = jnp.max(sc, axis=-1, keepdims=True)
        e = jnp.exp(sc - m)
        p = e / jnp.sum(e, axis=-1, keepdims=True)
        parts.append(_dg(p, vh))
    return jnp.concatenate(parts, axis=1)


def _silu(t):
    return t * (1.0 / (1.0 + jnp.exp(-t)))


def _ident(t):
    return t


GPB = 10
RW = GPB * NPG


def _proj_body(x_ref, qb, qs, kb, ks, vb, vs, q_ref, k_ref, v_ref):
    x = x_ref[...]
    q_ref[...] = _kan_in128(x, qb, qs, L1, 1, _ident)
    k_ref[...] = _kan_in128(x, kb, ks, L1, 1, _ident)
    v_ref[...] = _kan_in128(x, vb, vs, L1, 1, _ident)


def _attn_body(q_ref, k_ref, v_ref, o_ref):
    for g in range(GPB):
        o_ref[g] = _attn(q_ref[g], k_ref[g], v_ref[g])


def _final_body(x_ref, s_ref, dv_ref, ctx_ref,
                ob, osw, l1b, l1s, l2b, l2s, f1b, f1s, f2b, f2s,
                o_ref):
    x = x_ref[...]
    sagg = s_ref[0] + s_ref[1]
    dinv = dv_ref[...]
    agg = dinv * sagg + (dinv * dinv) * x

    t1 = _kan_in128(agg, l1b, l1s, K3, 3, _silu)
    h_local = _kan_in16(t1, l2b, l2s, K3, 3, _silu)

    h_attn = _kan_in128(ctx_ref[...], ob, osw, L1, 1, _ident)
    h1 = _ln(x + h_local)
    h = h1 + _ln(x + h_attn)
    t2 = _kan_in128(h, f1b, f1s, K3, 3, _silu)
    ffn = _kan_in16(t2, f2b, f2s, K3, 3, _silu)
    o_ref[...] = _ln(h + ffn)


def _whole(shape):
    nd = len(shape)
    return pl.BlockSpec(shape, lambda i, _nd=nd: (0,) * _nd)


_scale_call = pl.pallas_call(
    _scale_body,
    grid=(N // RB,),
    in_specs=[
        pl.BlockSpec((RB, D), lambda i: (i, 0)),
        pl.BlockSpec((NC, RB, 1), lambda i: (0, i, 0)),
    ],
    out_specs=[
        pl.BlockSpec((RB, D), lambda i: (i, 0)),
        pl.BlockSpec((RB, 1), lambda i: (i, 0)),
    ],
    out_shape=[
        jax.ShapeDtypeStruct((N, D), jnp.float32),
        jax.ShapeDtypeStruct((N, 1), jnp.float32),
    ],
)


_proj_call = pl.pallas_call(
    _proj_body,
    grid=(N // RW,),
    in_specs=[pl.BlockSpec((RW, D), lambda i: (i, 0))]
    + [_whole(shape) for shape in [(D, D), (3, D, D)] * 3],
    out_specs=[pl.BlockSpec((RW, D), lambda i: (i, 0))] * 3,
    out_shape=[jax.ShapeDtypeStruct((N, D), jnp.float32)] * 3,
)


_attn_call = pl.pallas_call(
    _attn_body,
    grid=(G // GPB,),
    in_specs=[pl.BlockSpec((GPB, NPG, D), lambda i: (i, 0, 0))] * 3,
    out_specs=pl.BlockSpec((GPB, NPG, D), lambda i: (i, 0, 0)),
    out_shape=jax.ShapeDtypeStruct((G, NPG, D), jnp.float32),
)


_final_call = pl.pallas_call(
    _final_body,
    grid=(N // RW,),
    in_specs=[
        pl.BlockSpec((RW, D), lambda i: (i, 0)),
        pl.BlockSpec((NC, RW, D), lambda i: (0, i, 0)),
        pl.BlockSpec((RW, 1), lambda i: (i, 0)),
        pl.BlockSpec((RW, D), lambda i: (i, 0)),
    ]
    + [_whole(shape) for shape in [(D, D), (3, D, D)]]
    + [_whole(shape) for shape in [(16, D), (7, 16, D), (16, D),
                                   (7, 16, D)] * 2],
    out_specs=pl.BlockSpec((RW, D), lambda i: (i, 0)),
    out_shape=jax.ShapeDtypeStruct((N, D), jnp.float32),
)


def _prep_weights(params):
    qkv, ow = [], []
    for name in ("q", "k", "v"):
        bw, sw = params[name]
        qkv += [bw, jnp.transpose(sw, (2, 0, 1))]
    bw, sw = params["o"]
    ow += [bw, jnp.transpose(sw, (2, 0, 1))]
    lf = []
    for name in ("local", "ffn"):
        (b1, s1), (b2, s2) = params[name]
        lf += [b1, jnp.transpose(s1, (2, 0, 1)),
               b2.T, jnp.transpose(s2, (2, 1, 0))]
    return qkv, ow, lf


def kernel(x, edge_index, batch, params):
    del batch
    ei = edge_index.astype(jnp.int32)
    row = ei[0].reshape(NW, NCH, 1, CK)
    col = ei[1].reshape(NW, NCH, 1, CK)
    z1 = jnp.zeros((1, N), jnp.float32)
    znd = jnp.zeros((N, D), jnp.float32)
    qkv, ow, lf = _prep_weights(params)
    deg_call, scat_call = _sc_calls()
    degp = deg_call(row, z1)
    degt = jnp.swapaxes(degp, 1, 2)
    y, dv = _scale_call(x, degt)
    sp = scat_call(y.reshape(1, N, D), row, col, znd)
    q, k, v = _proj_call(x, *qkv)
    ctx = _attn_call(q.reshape(G, NPG, D), k.reshape(G, NPG, D),
                     v.reshape(G, NPG, D))
    return _final_call(x, sp, dv, ctx.reshape(N, D), *ow, *lf)
